# baseline (speedup 1.0000x reference)
import jax
import jax.numpy as jnp
from jax.experimental import pallas as pl
from jax.experimental.pallas import tpu as pltpu

_N_FREQ = 12000
_N_RARE = 8000
_SHAPE = 20000
_BB = 32


def _assemble(ef, er, mf, mr, pf, pr, out):
    zeros = jnp.zeros((out.shape[0], _SHAPE - _N_FREQ), dtype=out.dtype)
    for m, (f, r) in enumerate(((ef, er), (mf, mr), (pf, pr))):
        base = m * _SHAPE
        out[:, base:base + _N_RARE] = r[...]
        out[:, base + _N_RARE:base + _N_FREQ] = f[:, _N_RARE:_N_FREQ]
        out[:, base + _N_FREQ:base + _SHAPE] = zeros


def kernel(esm_freq_out, esm_rare_out, msa_freq_out, msa_rare_out,
           interpro_freq_out, interpro_rare_out, freq_indicies, rare_indicies):
    batch = esm_freq_out.shape[0]
    freq_spec = pl.BlockSpec((_BB, _N_FREQ), lambda i: (i, 0))
    rare_spec = pl.BlockSpec((_BB, _N_RARE), lambda i: (i, 0))
    return pl.pallas_call(
        _assemble,
        grid=(batch // _BB,),
        in_specs=[freq_spec, rare_spec] * 3,
        out_specs=pl.BlockSpec((_BB, 3 * _SHAPE), lambda i: (i, 0)),
        out_shape=jax.ShapeDtypeStruct((batch, 3 * _SHAPE), esm_freq_out.dtype),
        compiler_params=pltpu.CompilerParams(
            dimension_semantics=("parallel",),
        ),
    )(esm_freq_out, esm_rare_out, msa_freq_out, msa_rare_out,
      interpro_freq_out, interpro_rare_out)
